# baseline (speedup 1.0000x reference)
import jax
import jax.numpy as jnp
from jax.experimental import pallas as pl
from jax.experimental.pallas import tpu as pltpu

N = 10000
K = 5000
D = 128
NCOPY = 10
ROWS_PER_COPY = N // NCOPY


def _body(idx_ref, g_ref, h_ref, g_out_ref, out_ref, sems):
    copies = [
        pltpu.make_async_copy(
            g_ref.at[pl.ds(i * ROWS_PER_COPY, ROWS_PER_COPY), :],
            g_out_ref.at[pl.ds(i * ROWS_PER_COPY, ROWS_PER_COPY), :],
            sems.at[i],
        )
        for i in range(NCOPY)
    ]
    for cp in copies:
        cp.start()

    out_ref[...] = jnp.zeros_like(out_ref)

    def loop(k, carry):
        ik = idx_ref[k]
        out_ref[pl.ds(ik, 1), :] = h_ref[pl.ds(k, 1), :]
        return carry

    jax.lax.fori_loop(0, K, loop, 0)

    for cp in copies:
        cp.wait()


def kernel(g, h, pre_h, idx):
    idx32 = idx.astype(jnp.int32)
    g_out, new_h = pl.pallas_call(
        _body,
        out_shape=(
            jax.ShapeDtypeStruct((N, N), jnp.float32),
            jax.ShapeDtypeStruct((N, D), jnp.float32),
        ),
        in_specs=[
            pl.BlockSpec(memory_space=pltpu.SMEM),
            pl.BlockSpec(memory_space=pl.ANY),
            pl.BlockSpec(memory_space=pltpu.VMEM),
        ],
        out_specs=(
            pl.BlockSpec(memory_space=pl.ANY),
            pl.BlockSpec(memory_space=pltpu.VMEM),
        ),
        scratch_shapes=[pltpu.SemaphoreType.DMA((NCOPY,))],
    )(idx32, g, h)
    return (g_out, new_h)


# speedup vs baseline: 48.2090x; 48.2090x over previous
import jax
import jax.numpy as jnp
from jax.experimental import pallas as pl
from jax.experimental.pallas import tpu as pltpu

N = 10000
K = 5000
D = 128
NBLK = 50
BR = N // NBLK
KC = K // NBLK


def _body(idx_ref, g_ref, h_ref, g_out_ref, out_ref):
    i = pl.program_id(0)

    @pl.when(i == 0)
    def _zero():
        out_ref[...] = jnp.zeros_like(out_ref)

    g_out_ref[...] = g_ref[...]

    def loop(k, carry):
        ik = idx_ref[k]
        out_ref[pl.ds(ik, 1), :] = h_ref[pl.ds(k, 1), :]
        return carry

    jax.lax.fori_loop(i * KC, (i + 1) * KC, loop, 0)


def kernel(g, h, pre_h, idx):
    idx32 = idx.astype(jnp.int32)
    g_out, new_h = pl.pallas_call(
        _body,
        grid=(NBLK,),
        out_shape=(
            jax.ShapeDtypeStruct((N, N), jnp.float32),
            jax.ShapeDtypeStruct((N, D), jnp.float32),
        ),
        in_specs=[
            pl.BlockSpec(memory_space=pltpu.SMEM),
            pl.BlockSpec((BR, N), lambda i: (i, 0)),
            pl.BlockSpec((K, D), lambda i: (0, 0)),
        ],
        out_specs=(
            pl.BlockSpec((BR, N), lambda i: (i, 0)),
            pl.BlockSpec((N, D), lambda i: (0, 0)),
        ),
    )(idx32, g, h)
    return (g_out, new_h)
